# Initial kernel scaffold; baseline (speedup 1.0000x reference)
import jax
import jax.numpy as jnp
from jax.experimental import pallas as pl


def kernel(input_, table):
    raise NotImplementedError("write your pallas kernel here")


# speedup vs baseline: 1.4825x; 1.4825x over previous
import functools

import jax
import jax.numpy as jnp
from jax import lax
from jax.experimental import pallas as pl
from jax.experimental.pallas import tpu as pltpu
from jax.experimental.pallas import tpu_sc as plsc

_B, _T, _E = 4096, 200, 32
_N = _B * _T
_NW = 32
_CH = 128
_RPW = _N // _NW // _CH
_K = 10
_G = _RPW // _K

_mesh = plsc.VectorSubcoreMesh(core_axis_name="c", subcore_axis_name="s")


@functools.partial(
    pl.kernel,
    out_type=jax.ShapeDtypeStruct((_N // _CH, _CH, _E), jnp.float32),
    mesh=_mesh,
    scratch_types=[
        pltpu.VMEM((_RPW, _CH), jnp.int32),
        pltpu.VMEM((_K, _CH, _E), jnp.float32),
        pltpu.SemaphoreType.DMA,
    ],
    compiler_params=pltpu.CompilerParams(use_tc_tiling_on_sc=False),
)
def _sc_gather(idx_hbm, table_hbm, out_hbm, idx_v, buf_v, gsem):
    wid = lax.axis_index("s") * 2 + lax.axis_index("c")
    rbase = wid * _RPW
    pltpu.sync_copy(idx_hbm.at[pl.ds(rbase, _RPW)], idx_v)

    @pl.loop(0, _G)
    def _group(g):
        row = g * _K
        copies = [
            pltpu.async_copy(
                table_hbm.at[idx_v.at[row + j]], buf_v.at[j], gsem
            )
            for j in range(_K)
        ]
        for c in copies:
            c.wait()
        pltpu.sync_copy(buf_v, out_hbm.at[pl.ds(rbase + row, _K)])


def kernel(input_, table):
    idx = input_.reshape(_N // _CH, _CH)
    out = _sc_gather(idx, table)
    return out.reshape(_B, _T, _E)


# speedup vs baseline: 1.5020x; 1.0132x over previous
import functools

import jax
import jax.numpy as jnp
from jax import lax
from jax.experimental import pallas as pl
from jax.experimental.pallas import tpu as pltpu
from jax.experimental.pallas import tpu_sc as plsc

_B, _T, _E = 4096, 200, 32
_N = _B * _T
_NW = 32
_CH = 128
_RPW = _N // _NW // _CH
_K = 10
_G = _RPW // _K

_mesh = plsc.VectorSubcoreMesh(core_axis_name="c", subcore_axis_name="s")


@functools.partial(
    pl.kernel,
    out_type=jax.ShapeDtypeStruct((_N // _CH, _CH, _E), jnp.float32),
    mesh=_mesh,
    scratch_types=[
        pltpu.VMEM((_RPW, _CH), jnp.int32),
        pltpu.VMEM((2, _K, _CH, _E), jnp.float32),
        pltpu.SemaphoreType.DMA,
        pltpu.SemaphoreType.DMA,
        pltpu.SemaphoreType.DMA,
        pltpu.SemaphoreType.DMA,
    ],
    compiler_params=pltpu.CompilerParams(use_tc_tiling_on_sc=False),
)
def _sc_gather(idx_hbm, table_hbm, out_hbm, idx_v, buf_v, g0, g1, o0, o1):
    wid = lax.axis_index("s") * 2 + lax.axis_index("c")
    rbase = wid * _RPW
    pltpu.sync_copy(idx_hbm.at[pl.ds(rbase, _RPW)], idx_v)

    def start_gather(g, slot, sem):
        for j in range(_K):
            pltpu.async_copy(
                table_hbm.at[idx_v.at[g * _K + j]], buf_v.at[slot, j], sem
            )

    def wait_gather(slot, sem):
        for j in range(_K):
            pltpu.make_async_copy(
                table_hbm.at[idx_v.at[j]], buf_v.at[slot, j], sem
            ).wait()

    start_gather(0, 0, g0)

    _G2 = _G // 2

    @pl.loop(0, _G2)
    def _pair(i):
        ga, gb = 2 * i, 2 * i + 1
        start_gather(gb, 1, g1)
        wait_gather(0, g0)
        sa = pltpu.async_copy(
            buf_v.at[0], out_hbm.at[pl.ds(rbase + ga * _K, _K)], o0
        )
        sa.wait()

        @pl.when(i < _G2 - 1)
        def _():
            start_gather(gb + 1, 0, g0)

        wait_gather(1, g1)
        sb = pltpu.async_copy(
            buf_v.at[1], out_hbm.at[pl.ds(rbase + gb * _K, _K)], o1
        )
        sb.wait()


def kernel(input_, table):
    idx = input_.reshape(_N // _CH, _CH)
    out = _sc_gather(idx, table)
    return out.reshape(_B, _T, _E)
